# baseline (speedup 1.0000x reference)
import jax
import jax.numpy as jnp
import numpy as np
from jax.experimental import pallas as pl
from jax.experimental.pallas import tpu as pltpu

_L = 10.0
_V0 = 0.05
_BETA = 1.0
_A_CF = 1.0
_A_CIL = 1.0
_R = 1.0
_A = 0.1
_D_MAC = 1.0
_N = 2048
_BM = 512
_NB = _N // _BM
_INV_L = 1.0 / _L
_RHO_R2 = (_R / _L) * (_R / _L)

_PAIRS = [(i, j) for i in range(_NB) for j in range(i + 1)]
_BI = np.array([p[0] for p in _PAIRS], dtype=np.int32)
_BJ = np.array([p[1] for p in _PAIRS], dtype=np.int32)
_NSTEPS = len(_PAIRS)


def _pair_kernel(bi_ref, bj_ref, y_ref, o_ref,
                 tab_ref, tabt_ref,
                 atx_ref, aty_ref, apx_ref, apy_ref, accr_ref):
    k = pl.program_id(0)
    bi = bi_ref[k]
    bj = bj_ref[k]

    @pl.when(k == 0)
    def _init():
        yt = y_ref[:, :].T
        xs = yt[0:1, :] * _INV_L
        ys = yt[1:2, :] * _INV_L
        th = yt[2:3, :]
        tabt = jnp.concatenate([xs, ys, jnp.cos(th), jnp.sin(th)], axis=0)
        tabt_ref[:, :] = tabt
        tab_ref[:, :] = tabt.T
        z = jnp.zeros((_N, 128), jnp.float32)
        atx_ref[:, :] = z
        aty_ref[:, :] = z
        apx_ref[:, :] = z
        apy_ref[:, :] = z
        accr_ref[:, :] = jnp.zeros((4, _N), jnp.float32)

    row = tab_ref[pl.ds(bi * _BM, _BM), :]
    col = tabt_ref[:, pl.ds(bj * _BM, _BM)]
    xi = row[:, 0:1]
    yi = row[:, 1:2]
    ci = row[:, 2:3]
    si = row[:, 3:4]
    xj = col[0:1, :]
    yj = col[1:2, :]
    cj = col[2:3, :]
    sj = col[3:4, :]

    u = xi - xj
    u = u - jnp.round(u)
    v = yi - yj
    v = v - jnp.round(v)

    rho2 = u * u + v * v + 1e-14
    m = jnp.where(rho2 < _RHO_R2, 1.0, 0.0)
    q = m * jax.lax.rsqrt(rho2)
    wcil = q - _L * m
    rho = rho2 * q
    wdiff = q * jnp.exp(rho * -_L) - wcil
    pcx = u * wcil
    pcy = v * wcil
    txi = m * cj + u * wdiff
    tyi = m * sj + v * wdiff

    def _fold(f):
        return (f[:, 0:128] + f[:, 128:256]) + (f[:, 256:384] + f[:, 384:512])

    sl = pl.ds(bi * _BM, _BM)
    atx_ref[sl, :] += _fold(txi)
    aty_ref[sl, :] += _fold(tyi)
    apx_ref[sl, :] += _fold(pcx)
    apy_ref[sl, :] += _fold(pcy)

    @pl.when(bi != bj)
    def _other_side():
        txj = m * ci - u * wdiff
        tyj = m * si - v * wdiff
        row = jnp.concatenate(
            [
                jnp.sum(txj, axis=0, keepdims=True),
                jnp.sum(tyj, axis=0, keepdims=True),
                -jnp.sum(pcx, axis=0, keepdims=True),
                -jnp.sum(pcy, axis=0, keepdims=True),
            ],
            axis=0,
        )
        accr_ref[:, pl.ds(bj * _BM, _BM)] += row

    @pl.when(k == _NSTEPS - 1)
    def _finish():
        def _colsum(aref):
            return jnp.sum(aref[:, :].T, axis=0, keepdims=True)

        sx = _colsum(atx_ref) + accr_ref[0:1, :]
        sy = _colsum(aty_ref) + accr_ref[1:2, :]
        cx = _colsum(apx_ref) + accr_ref[2:3, :]
        cy = _colsum(apy_ref) + accr_ref[3:4, :]
        cif = tabt_ref[2:3, :]
        sif = tabt_ref[3:4, :]
        dth = cif * sy - sif * sx + _A * cif
        ox = _V0 * cif - _BETA * cx
        oy = _V0 * sif - _BETA * cy
        o_ref[:, :] = jnp.concatenate([ox, oy, dth], axis=0).T


@jax.jit
def _drift(y):
    return pl.pallas_call(
        _pair_kernel,
        grid_spec=pltpu.PrefetchScalarGridSpec(
            num_scalar_prefetch=2,
            grid=(_NSTEPS,),
            in_specs=[
                pl.BlockSpec((_N, 3), lambda k, bi, bj: (0, 0)),
            ],
            out_specs=pl.BlockSpec((_N, 3), lambda k, bi, bj: (0, 0)),
            scratch_shapes=[
                pltpu.VMEM((_N, 4), jnp.float32),
                pltpu.VMEM((4, _N), jnp.float32),
                pltpu.VMEM((_N, 128), jnp.float32),
                pltpu.VMEM((_N, 128), jnp.float32),
                pltpu.VMEM((_N, 128), jnp.float32),
                pltpu.VMEM((_N, 128), jnp.float32),
                pltpu.VMEM((4, _N), jnp.float32),
            ],
        ),
        out_shape=jax.ShapeDtypeStruct((_N, 3), jnp.float32),
        compiler_params=pltpu.CompilerParams(
            dimension_semantics=("arbitrary",),
        ),
    )(jnp.asarray(_BI), jnp.asarray(_BJ), y)


def kernel(t, y):
    return _drift(y)


# speedup vs baseline: 1.0054x; 1.0054x over previous
import jax
import jax.numpy as jnp
import numpy as np
from jax.experimental import pallas as pl
from jax.experimental.pallas import tpu as pltpu

_L = 10.0
_V0 = 0.05
_BETA = 1.0
_A_CF = 1.0
_A_CIL = 1.0
_R = 1.0
_A = 0.1
_D_MAC = 1.0
_N = 2048
_BM = 512
_NB = _N // _BM
_INV_L = 1.0 / _L
_RHO_R2 = (_R / _L) * (_R / _L)

_PAIRS = [(i, j) for i in range(_NB) for j in range(i + 1)]
_BI = np.array([p[0] for p in _PAIRS], dtype=np.int32)
_BJ = np.array([p[1] for p in _PAIRS], dtype=np.int32)
_NSTEPS = len(_PAIRS)


def _table_kernel(yt_ref, tab_ref, tabt_ref):
    xs = yt_ref[0:1, :] * _INV_L
    ys = yt_ref[1:2, :] * _INV_L
    th = yt_ref[2:3, :]
    c = jnp.cos(th)
    s = jnp.sin(th)
    tabt = jnp.concatenate([xs, ys, c, s], axis=0)
    tabt_ref[:, :] = tabt
    tab_ref[:, :] = tabt.T


def _pair_kernel(bi_ref, bj_ref, row_ref, col_ref, tabf_ref, o_ref,
                 atx_ref, aty_ref, apx_ref, apy_ref, accr_ref):
    k = pl.program_id(0)
    bi = bi_ref[k]
    bj = bj_ref[k]

    @pl.when(k == 0)
    def _init():
        accr_ref[:, pl.ds((_NB - 1) * _BM, _BM)] = jnp.zeros(
            (4, _BM), jnp.float32)

    xi = row_ref[:, 0:1]
    yi = row_ref[:, 1:2]
    ci = row_ref[:, 2:3]
    si = row_ref[:, 3:4]
    xj = col_ref[0:1, :]
    yj = col_ref[1:2, :]
    cj = col_ref[2:3, :]
    sj = col_ref[3:4, :]

    u = xi - xj
    u = u - jnp.round(u)
    v = yi - yj
    v = v - jnp.round(v)

    rho2 = u * u + v * v + 1e-14
    m = jnp.where(rho2 < _RHO_R2, 1.0, 0.0)
    q = m * jax.lax.rsqrt(rho2)
    wcil = q - _L * m
    rho = rho2 * q
    wdiff = q * jnp.exp(rho * -_L) - wcil
    pcx = u * wcil
    pcy = v * wcil
    fx = u * wdiff
    fy = v * wdiff
    txi = m * cj + fx
    tyi = m * sj + fy

    def _fold(f):
        return (f[:, 0:128] + f[:, 128:256]) + (f[:, 256:384] + f[:, 384:512])

    sl = pl.ds(bi * _BM, _BM)
    ftx = _fold(txi)
    fty = _fold(tyi)
    fpx = _fold(pcx)
    fpy = _fold(pcy)

    @pl.when(bj == 0)
    def _store_first():
        atx_ref[sl, :] = ftx
        aty_ref[sl, :] = fty
        apx_ref[sl, :] = fpx
        apy_ref[sl, :] = fpy

    @pl.when(bj != 0)
    def _store_acc():
        atx_ref[sl, :] += ftx
        aty_ref[sl, :] += fty
        apx_ref[sl, :] += fpx
        apy_ref[sl, :] += fpy

    @pl.when(bi != bj)
    def _other_side():
        txj = m * ci - fx
        tyj = m * si - fy
        row = jnp.concatenate(
            [
                jnp.sum(txj, axis=0, keepdims=True),
                jnp.sum(tyj, axis=0, keepdims=True),
                -jnp.sum(pcx, axis=0, keepdims=True),
                -jnp.sum(pcy, axis=0, keepdims=True),
            ],
            axis=0,
        )
        slj = pl.ds(bj * _BM, _BM)

        @pl.when(bi == bj + 1)
        def _row_first():
            accr_ref[:, slj] = row

        @pl.when(bi != bj + 1)
        def _row_acc():
            accr_ref[:, slj] += row

    @pl.when(k == _NSTEPS - 1)
    def _finish():
        def _colsum(aref):
            return jnp.sum(aref[:, :].T, axis=0, keepdims=True)

        sx = _colsum(atx_ref) + accr_ref[0:1, :]
        sy = _colsum(aty_ref) + accr_ref[1:2, :]
        cx = _colsum(apx_ref) + accr_ref[2:3, :]
        cy = _colsum(apy_ref) + accr_ref[3:4, :]
        cif = tabf_ref[2:3, :]
        sif = tabf_ref[3:4, :]
        dth = cif * sy - sif * sx + _A * cif
        ox = _V0 * cif - _BETA * cx
        oy = _V0 * sif - _BETA * cy
        o_ref[:, :] = jnp.concatenate([ox, oy, dth], axis=0).T


@jax.jit
def _drift(y):
    tab, tabt = pl.pallas_call(
        _table_kernel,
        out_shape=(
            jax.ShapeDtypeStruct((_N, 4), jnp.float32),
            jax.ShapeDtypeStruct((4, _N), jnp.float32),
        ),
    )(y.T)
    return pl.pallas_call(
        _pair_kernel,
        grid_spec=pltpu.PrefetchScalarGridSpec(
            num_scalar_prefetch=2,
            grid=(_NSTEPS,),
            in_specs=[
                pl.BlockSpec((_BM, 4), lambda k, bi, bj: (bi[k], 0)),
                pl.BlockSpec((4, _BM), lambda k, bi, bj: (0, bj[k])),
                pl.BlockSpec((4, _N), lambda k, bi, bj: (0, 0)),
            ],
            out_specs=pl.BlockSpec((_N, 3), lambda k, bi, bj: (0, 0)),
            scratch_shapes=[
                pltpu.VMEM((_N, 128), jnp.float32),
                pltpu.VMEM((_N, 128), jnp.float32),
                pltpu.VMEM((_N, 128), jnp.float32),
                pltpu.VMEM((_N, 128), jnp.float32),
                pltpu.VMEM((4, _N), jnp.float32),
            ],
        ),
        out_shape=jax.ShapeDtypeStruct((_N, 3), jnp.float32),
        compiler_params=pltpu.CompilerParams(
            dimension_semantics=("arbitrary",),
        ),
    )(jnp.asarray(_BI), jnp.asarray(_BJ), tab, tabt, tabt)


def kernel(t, y):
    return _drift(y)


# speedup vs baseline: 1.0319x; 1.0263x over previous
import jax
import jax.numpy as jnp
import numpy as np
from jax.experimental import pallas as pl
from jax.experimental.pallas import tpu as pltpu

_L = 10.0
_V0 = 0.05
_BETA = 1.0
_A_CF = 1.0
_A_CIL = 1.0
_R = 1.0
_A = 0.1
_D_MAC = 1.0
_N = 2048
_BM = 512
_NB = _N // _BM
_INV_L = 1.0 / _L
_RHO_R2 = (_R / _L) * (_R / _L)

_PAIRS = [(i, j) for i in range(_NB) for j in range(i + 1)]
_BI = np.array([p[0] for p in _PAIRS], dtype=np.int32)
_BJ = np.array([p[1] for p in _PAIRS], dtype=np.int32)
_NSTEPS = len(_PAIRS)


def _table_kernel(yt_ref, tab_ref, tabt_ref):
    xs = yt_ref[0:1, :] * _INV_L
    ys = yt_ref[1:2, :] * _INV_L
    th = yt_ref[2:3, :]
    c = jnp.cos(th)
    s = jnp.sin(th)
    tabt = jnp.concatenate([xs, ys, c, s], axis=0)
    tabt_ref[:, :] = tabt
    tab_ref[:, :] = tabt.T


def _pair_kernel(bi_ref, bj_ref, row_ref, col_ref, tabf_ref, o_ref,
                 atx_ref, aty_ref, apx_ref, apy_ref, accr_ref):
    k = pl.program_id(0)
    bi = bi_ref[k]
    bj = bj_ref[k]

    @pl.when(k == 0)
    def _init():
        z = jnp.zeros((_N, 128), jnp.float32)
        atx_ref[:, :] = z
        aty_ref[:, :] = z
        apx_ref[:, :] = z
        apy_ref[:, :] = z
        accr_ref[:, :] = jnp.zeros((4, _N), jnp.float32)

    xi = row_ref[:, 0:1]
    yi = row_ref[:, 1:2]
    ci = row_ref[:, 2:3]
    si = row_ref[:, 3:4]
    xj = col_ref[0:1, :]
    yj = col_ref[1:2, :]
    cj = col_ref[2:3, :]
    sj = col_ref[3:4, :]

    u = xi - xj
    u = u - jnp.round(u)
    v = yi - yj
    v = v - jnp.round(v)

    rho2 = u * u + v * v + 1e-14
    m = jnp.where(rho2 < _RHO_R2, 1.0, 0.0)
    q = m * jax.lax.rsqrt(rho2)
    wcil = q - _L * m
    rho = rho2 * q
    wdiff = q * jnp.exp(rho * -_L) - wcil
    pcx = u * wcil
    pcy = v * wcil
    fx = u * wdiff
    fy = v * wdiff
    txi = m * cj + fx
    tyi = m * sj + fy

    def _fold(f):
        return (f[:, 0:128] + f[:, 128:256]) + (f[:, 256:384] + f[:, 384:512])

    sl = pl.ds(bi * _BM, _BM)
    atx_ref[sl, :] += _fold(txi)
    aty_ref[sl, :] += _fold(tyi)
    apx_ref[sl, :] += _fold(pcx)
    apy_ref[sl, :] += _fold(pcy)

    @pl.when(bi != bj)
    def _other_side():
        txj = m * ci - fx
        tyj = m * si - fy
        row = jnp.concatenate(
            [
                jnp.sum(txj, axis=0, keepdims=True),
                jnp.sum(tyj, axis=0, keepdims=True),
                -jnp.sum(pcx, axis=0, keepdims=True),
                -jnp.sum(pcy, axis=0, keepdims=True),
            ],
            axis=0,
        )
        accr_ref[:, pl.ds(bj * _BM, _BM)] += row

    @pl.when(k == _NSTEPS - 1)
    def _finish():
        def _colsum(aref):
            return jnp.sum(aref[:, :].T, axis=0, keepdims=True)

        sx = _colsum(atx_ref) + accr_ref[0:1, :]
        sy = _colsum(aty_ref) + accr_ref[1:2, :]
        cx = _colsum(apx_ref) + accr_ref[2:3, :]
        cy = _colsum(apy_ref) + accr_ref[3:4, :]
        cif = tabf_ref[2:3, :]
        sif = tabf_ref[3:4, :]
        dth = cif * sy - sif * sx + _A * cif
        ox = _V0 * cif - _BETA * cx
        oy = _V0 * sif - _BETA * cy
        o_ref[:, :] = jnp.concatenate([ox, oy, dth], axis=0).T


@jax.jit
def _drift(y):
    tab, tabt = pl.pallas_call(
        _table_kernel,
        out_shape=(
            jax.ShapeDtypeStruct((_N, 4), jnp.float32),
            jax.ShapeDtypeStruct((4, _N), jnp.float32),
        ),
    )(y.T)
    return pl.pallas_call(
        _pair_kernel,
        grid_spec=pltpu.PrefetchScalarGridSpec(
            num_scalar_prefetch=2,
            grid=(_NSTEPS,),
            in_specs=[
                pl.BlockSpec((_BM, 4), lambda k, bi, bj: (bi[k], 0)),
                pl.BlockSpec((4, _BM), lambda k, bi, bj: (0, bj[k])),
                pl.BlockSpec((4, _N), lambda k, bi, bj: (0, 0)),
            ],
            out_specs=pl.BlockSpec((_N, 3), lambda k, bi, bj: (0, 0)),
            scratch_shapes=[
                pltpu.VMEM((_N, 128), jnp.float32),
                pltpu.VMEM((_N, 128), jnp.float32),
                pltpu.VMEM((_N, 128), jnp.float32),
                pltpu.VMEM((_N, 128), jnp.float32),
                pltpu.VMEM((4, _N), jnp.float32),
            ],
        ),
        out_shape=jax.ShapeDtypeStruct((_N, 3), jnp.float32),
        compiler_params=pltpu.CompilerParams(
            dimension_semantics=("arbitrary",),
        ),
    )(jnp.asarray(_BI), jnp.asarray(_BJ), tab, tabt, tabt)


def kernel(t, y):
    return _drift(y)
